# baseline (speedup 1.0000x reference)
import functools

import jax
import jax.numpy as jnp
from jax import lax
from jax.experimental import pallas as pl
from jax.experimental.pallas import tpu as pltpu
from jax.experimental.pallas import tpu_sc as plsc

N, E, V, D = 10000, 320000, 100000, 128
NC, NS, LN = 2, 16, 16
NW = NC * NS
NP = 10240
NB_N = NP // 128
EP = 327680
KB = EP // (NW * 128)
RPT = NP // NS
F32 = jnp.float32

_mesh = functools.partial(
    plsc.VectorSubcoreMesh, core_axis_name="c", subcore_axis_name="s")
_SC_PARAMS = pltpu.CompilerParams(needs_layout_passes=False)


def _wid():
    return lax.axis_index("c") * NS + lax.axis_index("s")


def _emb_body(x2, emb, h_out, idx_v, rows_v, sem):
    w = _wid()
    for t in range(3):
        b = w + t * NW

        @pl.when(b < NB_N)
        def _():
            pltpu.sync_copy(x2.at[b], idx_v)
            pltpu.async_copy(emb.at[idx_v], rows_v, sem).wait()
            pltpu.sync_copy(rows_v, h_out.at[pl.ds(b * 128, 128)])


def _emb_gather(x2, emb):
    return pl.kernel(
        _emb_body,
        out_type=jax.ShapeDtypeStruct((NP, D), F32),
        mesh=_mesh(),
        compiler_params=_SC_PARAMS,
        scratch_types=[
            pltpu.VMEM((128,), jnp.int32),
            pltpu.VMEM((128, D), F32),
            pltpu.SemaphoreType.DMA,
        ],
    )(x2, emb)


def _edgew_body(row3, col3, huv, w2p, zn, ew4, degp, ridx, cidx, u_v, v_v,
                ew_o, w2_v, deg_v, sem):
    w = _wid()
    pltpu.sync_copy(row3.at[w], ridx)
    pltpu.sync_copy(col3.at[w], cidx)
    pltpu.sync_copy(w2p, w2_v)
    pltpu.sync_copy(zn, deg_v)
    lanes = jax.lax.iota(jnp.int32, LN)
    b2sp = plsc.load_gather(w2_v, [jnp.full((LN,), 64, jnp.int32)])
    ebase0 = w * (KB * 128)

    def body(j, carry):
        cp1 = pltpu.async_copy(huv.at[ridx.at[j]], u_v, sem)
        cp2 = pltpu.async_copy(huv.at[cidx.at[j]], v_v, sem)
        cp1.wait()
        cp2.wait()
        evs = [g * LN + lanes for g in range(8)]

        def dot_k(k, accs):
            wsp = plsc.load_gather(w2_v, [jnp.full((LN,), k, jnp.int32)])
            kk = jnp.full((LN,), k, jnp.int32)
            kk2 = jnp.full((LN,), 64 + k, jnp.int32)
            return tuple(
                acc + wsp * jnp.maximum(
                    plsc.load_gather(u_v, [evs[g], kk])
                    + plsc.load_gather(v_v, [evs[g], kk2]), 0.0)
                for g, acc in enumerate(accs))

        accs = lax.fori_loop(0, 64, dot_k, tuple(jnp.zeros((LN,), F32)
                                                 for _ in range(8)),
                             unroll=4)
        ebase = ebase0 + j * 128
        for g in range(8):
            t = accs[g] + b2sp
            sig = 1.0 / (1.0 + jnp.exp(-t))
            eid = ebase + g * LN + lanes
            ew16 = jnp.where(eid < E, sig, 0.0)
            ew_o[j, pl.ds(g * LN, LN)] = ew16
            col16 = cidx[j, pl.ds(g * LN, LN)]
            plsc.addupdate_scatter(deg_v, [col16], ew16)
        return carry

    lax.fori_loop(0, KB, body, 0)
    pltpu.sync_copy(ew_o, ew4.at[w])
    pltpu.sync_copy(deg_v, degp.at[w])


def _edge_w_deg(row3, col3, huv, w2p, zn):
    return pl.kernel(
        _edgew_body,
        out_type=[
            jax.ShapeDtypeStruct((NW, KB, 128), F32),
            jax.ShapeDtypeStruct((NW, NP), F32),
        ],
        mesh=_mesh(),
        compiler_params=_SC_PARAMS,
        scratch_types=[
            pltpu.VMEM((KB, 128), jnp.int32),
            pltpu.VMEM((KB, 128), jnp.int32),
            pltpu.VMEM((128, D), F32),
            pltpu.VMEM((128, D), F32),
            pltpu.VMEM((KB, 128), F32),
            pltpu.VMEM((128,), F32),
            pltpu.VMEM((NP,), F32),
            pltpu.SemaphoreType.DMA,
        ],
    )(row3, col3, huv, w2p, zn)


def _conv_body(row3, col3, ew3, y, zrows, out2, ridx, cidx, ew_v, rows_v,
               acc, sem):
    c = lax.axis_index("c")
    s = lax.axis_index("s")
    w = c * NS + s
    pltpu.sync_copy(zrows.at[pl.ds(s * RPT, RPT)], acc.at[pl.ds(s * RPT, RPT)])
    pltpu.sync_copy(row3.at[w], ridx)
    pltpu.sync_copy(col3.at[w], cidx)
    pltpu.sync_copy(ew3.at[w], ew_v)
    plsc.subcore_barrier()

    def body(j, carry):
        pltpu.async_copy(y.at[ridx.at[j]], rows_v, sem).wait()
        jsp = jnp.full((LN,), j, jnp.int32)

        def scale(e, c2):
            esp = jnp.full((LN,), e, jnp.int32)
            wsp = plsc.load_gather(ew_v, [jsp, esp])
            for k in range(D // LN):
                rows_v[e, pl.ds(k * LN, LN)] = (
                    rows_v[e, pl.ds(k * LN, LN)] * wsp)
            return c2

        lax.fori_loop(0, 128, scale, 0, unroll=2)
        pltpu.sync_copy(rows_v, acc.at[cidx.at[j]], add=True)
        return carry

    lax.fori_loop(0, KB, body, 0)
    plsc.subcore_barrier()
    pltpu.sync_copy(acc.at[pl.ds(s * RPT, RPT)],
                    out2.at[c, pl.ds(s * RPT, RPT)])


def _conv_agg(row3, col3, ew3, y, zrows):
    return pl.kernel(
        _conv_body,
        out_type=jax.ShapeDtypeStruct((NC, NP, D), F32),
        mesh=_mesh(),
        compiler_params=_SC_PARAMS,
        scratch_types=[
            pltpu.VMEM((KB, 128), jnp.int32),
            pltpu.VMEM((KB, 128), jnp.int32),
            pltpu.VMEM((KB, 128), F32),
            pltpu.VMEM((128, D), F32),
            pltpu.VMEM_SHARED((NP, D), F32),
            pltpu.SemaphoreType.DMA,
        ],
    )(row3, col3, ew3, y, zrows)


def _mm_body(h_ref, w1c, b1c, wg1, huv_ref, xw_ref):
    hb = h_ref[...]
    huv_ref[...] = jnp.dot(hb, w1c[...], preferred_element_type=F32) + b1c[...]
    xw_ref[...] = jnp.dot(hb, wg1[...], preferred_element_type=F32)


def _node_mm(h, w1c, b1c, wg1):
    rb = 1024
    grid = NP // rb
    full = lambda shp: pl.BlockSpec(shp, lambda i: (0, 0))
    return pl.pallas_call(
        _mm_body,
        grid=grid,
        in_specs=[
            pl.BlockSpec((rb, D), lambda i: (i, 0)),
            full((D, D)), full((1, D)), full((D, D)),
        ],
        out_specs=[
            pl.BlockSpec((rb, D), lambda i: (i, 0)),
            pl.BlockSpec((rb, D), lambda i: (i, 0)),
        ],
        out_shape=[
            jax.ShapeDtypeStruct((NP, D), F32),
            jax.ShapeDtypeStruct((NP, D), F32),
        ],
    )(h, w1c, b1c, wg1)


def _dinv_body(degp_ref, xw_ref, dinv_ref, y1_ref):
    i = pl.program_id(0)
    deg = jnp.sum(degp_ref[...], axis=0)[:, None] + 1.0
    nidx = i * 1024 + lax.broadcasted_iota(jnp.int32, (1024, 1), 0)
    dinv = jnp.where(nidx < N, lax.rsqrt(deg), 0.0)
    dinv_ref[...] = dinv
    y1_ref[...] = dinv * xw_ref[...]


def _dinv_y1(degp, xw1):
    grid = NP // 1024
    return pl.pallas_call(
        _dinv_body,
        grid=grid,
        in_specs=[
            pl.BlockSpec((NW, 1024), lambda i: (0, i)),
            pl.BlockSpec((1024, D), lambda i: (i, 0)),
        ],
        out_specs=[
            pl.BlockSpec((1024, 1), lambda i: (i, 0)),
            pl.BlockSpec((1024, D), lambda i: (i, 0)),
        ],
        out_shape=[
            jax.ShapeDtypeStruct((NP, 1), F32),
            jax.ShapeDtypeStruct((NP, D), F32),
        ],
    )(degp, xw1)


def _post_body(p_ref, dinv_ref, xw_ref, bg_ref, wg2_ref, xw2_ref, y2_ref):
    dinv = dinv_ref[...]
    acc = p_ref[0] + p_ref[1]
    h1 = jnp.maximum(dinv * acc + dinv * dinv * xw_ref[...] + bg_ref[...], 0.0)
    xw2 = jnp.dot(h1, wg2_ref[...], preferred_element_type=F32)
    xw2_ref[...] = xw2
    y2_ref[...] = dinv * xw2


def _post_conv1(p, dinv, xw1, bgr, wg2):
    grid = NP // 1024
    return pl.pallas_call(
        _post_body,
        grid=grid,
        in_specs=[
            pl.BlockSpec((NC, 1024, D), lambda i: (0, i, 0)),
            pl.BlockSpec((1024, 1), lambda i: (i, 0)),
            pl.BlockSpec((1024, D), lambda i: (i, 0)),
            pl.BlockSpec((1, D), lambda i: (0, 0)),
            pl.BlockSpec((D, D), lambda i: (0, 0)),
        ],
        out_specs=[
            pl.BlockSpec((1024, D), lambda i: (i, 0)),
            pl.BlockSpec((1024, D), lambda i: (i, 0)),
        ],
        out_shape=[
            jax.ShapeDtypeStruct((NP, D), F32),
            jax.ShapeDtypeStruct((NP, D), F32),
        ],
    )(p, dinv, xw1, bgr, wg2)


def _final_body(p_ref, dinv_ref, xw2_ref, bg2_ref, wfc_ref, bfc_ref, out_ref):
    dinv = dinv_ref[...]
    acc = p_ref[0] + p_ref[1]
    h2 = jnp.maximum(dinv * acc + dinv * dinv * xw2_ref[...] + bg2_ref[...],
                     0.0)
    nidx = lax.broadcasted_iota(jnp.int32, (NP, 1), 0)
    pooled = jnp.sum(jnp.where(nidx < N, h2, 0.0), axis=0, keepdims=True) / N
    out_ref[...] = jax.nn.sigmoid(
        jnp.dot(pooled, wfc_ref[...], preferred_element_type=F32)
        + bfc_ref[...])


def _final(p2, dinv, xw2, bg2r, wfc, bfcr):
    return pl.pallas_call(
        _final_body,
        out_shape=jax.ShapeDtypeStruct((1, 1), F32),
    )(p2, dinv, xw2, bg2r, wfc, bfcr)


def kernel(x, edge_index, emb_table, W1, b1, W2, b2, Wg1, bg1, Wg2, bg2,
           Wfc, bfc):
    x = x.astype(jnp.int32)
    row = edge_index[0].astype(jnp.int32)
    col = edge_index[1].astype(jnp.int32)
    x2 = jnp.concatenate([x, jnp.zeros((NP - N,), jnp.int32)]).reshape(
        NB_N, 128)
    zpad = jnp.zeros((EP - E,), jnp.int32)
    row3 = jnp.concatenate([row, zpad]).reshape(NW, KB, 128)
    col3 = jnp.concatenate([col, zpad]).reshape(NW, KB, 128)
    zn = jnp.zeros((NP,), F32)
    zrows = jnp.zeros((NP, D), F32)

    h = _emb_gather(x2, emb_table)
    w1c = jnp.concatenate([W1[:D], W1[D:]], axis=1)
    b1c = jnp.concatenate([jnp.zeros((64,), F32), b1]).reshape(1, D)
    huv, xw1 = _node_mm(h, w1c, b1c, Wg1)
    w2p = jnp.concatenate(
        [W2[:, 0], b2, jnp.zeros((63,), F32)]).astype(F32)
    ew3, degp = _edge_w_deg(row3, col3, huv, w2p, zn)
    dinv, y1 = _dinv_y1(degp, xw1)
    p1 = _conv_agg(row3, col3, ew3, y1, zrows)
    xw2, y2 = _post_conv1(p1, dinv, xw1, bg1.reshape(1, D), Wg2)
    p2 = _conv_agg(row3, col3, ew3, y2, zrows)
    out = _final(p2, dinv, xw2, bg2.reshape(1, D), Wfc, bfc.reshape(1, 1))
    return out.reshape(1)


# speedup vs baseline: 1.2251x; 1.2251x over previous
import functools

import jax
import jax.numpy as jnp
from jax import lax
from jax.experimental import pallas as pl
from jax.experimental.pallas import tpu as pltpu
from jax.experimental.pallas import tpu_sc as plsc

N, E, V, D = 10000, 320000, 100000, 128
NC, NS, LN = 2, 16, 16
NW = NC * NS
NP = 10240
NB_N = NP // 128
EP = 327680
KB = EP // (NW * 128)
EW_ = KB * 128
BS = 256
NBL = EW_ // BS
GC = 16
RPT = NP // NS
F32 = jnp.float32

_mesh = functools.partial(
    plsc.VectorSubcoreMesh, core_axis_name="c", subcore_axis_name="s")
_SC_PARAMS = pltpu.CompilerParams(needs_layout_passes=False)


def _wid():
    return lax.axis_index("c") * NS + lax.axis_index("s")


def _emb_body(x2, emb, h_out, idx_v, rows_v, sem):
    w = _wid()
    for t in range(3):
        b = w + t * NW

        @pl.when(b < NB_N)
        def _():
            pltpu.sync_copy(x2.at[b], idx_v)
            pltpu.async_copy(emb.at[idx_v], rows_v, sem).wait()
            pltpu.sync_copy(rows_v, h_out.at[pl.ds(b * 128, 128)])


def _emb_gather(x2, emb):
    return pl.kernel(
        _emb_body,
        out_type=jax.ShapeDtypeStruct((NP, D), F32),
        mesh=_mesh(),
        compiler_params=_SC_PARAMS,
        scratch_types=[
            pltpu.VMEM((128,), jnp.int32),
            pltpu.VMEM((128, D), F32),
            pltpu.SemaphoreType.DMA,
        ],
    )(x2, emb)


def _edgeq_body(rowf, colf, huv, q4, ridx, cidx, u_v, v_v, q_v, sem):
    w = _wid()
    pltpu.sync_copy(rowf.at[w], ridx)
    pltpu.sync_copy(colf.at[w], cidx)

    def body(j, carry):
        cp1 = pltpu.async_copy(huv.at[ridx.at[pl.ds(j * BS, BS)]], u_v, sem)
        cp2 = pltpu.async_copy(huv.at[cidx.at[pl.ds(j * BS, BS)]], v_v, sem)
        cp1.wait()
        cp2.wait()

        def addhalf(e, c2):
            for k in range(64 // LN):
                q_v[e, pl.ds(k * LN, LN)] = (
                    u_v[e, pl.ds(k * LN, LN)]
                    + v_v[e, pl.ds(64 + k * LN, LN)])
            return c2

        lax.fori_loop(0, BS, addhalf, 0)
        pltpu.sync_copy(q_v, q4.at[w, j])
        return carry

    lax.fori_loop(0, NBL, body, 0)


def _edge_q(rowf, colf, huv):
    return pl.kernel(
        _edgeq_body,
        out_type=jax.ShapeDtypeStruct((NW, NBL, BS, 64), F32),
        mesh=_mesh(),
        compiler_params=_SC_PARAMS,
        scratch_types=[
            pltpu.VMEM((EW_,), jnp.int32),
            pltpu.VMEM((EW_,), jnp.int32),
            pltpu.VMEM((BS, D), F32),
            pltpu.VMEM((BS, D), F32),
            pltpu.VMEM((BS, 64), F32),
            pltpu.SemaphoreType.DMA,
        ],
    )(rowf, colf, huv)


def _deg_body(col3, ew3, zn, degp, cidx, ew_v, deg_v):
    w = _wid()
    pltpu.sync_copy(zn, deg_v)
    pltpu.sync_copy(col3.at[w], cidx)
    pltpu.sync_copy(ew3.at[w], ew_v)

    def body(j, carry):
        def inner(g, c2):
            idx16 = cidx[j, pl.ds(g * LN, LN)]
            w16 = ew_v[j, pl.ds(g * LN, LN)]
            plsc.addupdate_scatter(deg_v, [idx16], w16)
            return c2
        return lax.fori_loop(0, 128 // LN, inner, carry)

    lax.fori_loop(0, KB, body, 0)
    pltpu.sync_copy(deg_v, degp.at[w])


def _deg_partials(col3, ew3, zn):
    return pl.kernel(
        _deg_body,
        out_type=jax.ShapeDtypeStruct((NW, NP), F32),
        mesh=_mesh(),
        compiler_params=_SC_PARAMS,
        scratch_types=[
            pltpu.VMEM((KB, 128), jnp.int32),
            pltpu.VMEM((KB, 128), F32),
            pltpu.VMEM((NP,), F32),
        ],
    )(col3, ew3, zn)


def _conv_body(rowf, col3, ew3, y, zrows, out2, ridx, cidx, ew_v, rows_v,
               acc, sem):
    c = lax.axis_index("c")
    s = lax.axis_index("s")
    w = c * NS + s
    pltpu.sync_copy(zrows.at[pl.ds(s * RPT, RPT)], acc.at[pl.ds(s * RPT, RPT)])
    pltpu.sync_copy(rowf.at[w], ridx)
    plsc.subcore_barrier()

    def scale_half(jj, base):
        jsp = jnp.full((LN,), jj, jnp.int32)

        def scale(e, c2):
            esp = jnp.full((LN,), e, jnp.int32)
            wsp = plsc.load_gather(ew_v, [jsp, esp])
            for k in range(D // LN):
                rows_v[base + e, pl.ds(k * LN, LN)] = (
                    rows_v[base + e, pl.ds(k * LN, LN)] * wsp)
            return c2

        lax.fori_loop(0, 128, scale, 0)

    def super_block(sg, carry):
        pltpu.sync_copy(col3.at[w, pl.ds(sg * GC, GC)], cidx)
        pltpu.sync_copy(ew3.at[w, pl.ds(sg * GC, GC)], ew_v)

        def body(jb, c2):
            off = (sg * GC + 2 * jb) * 128
            pltpu.async_copy(y.at[ridx.at[pl.ds(off, BS)]], rows_v,
                             sem).wait()
            scale_half(2 * jb, 0)
            scale_half(2 * jb + 1, 128)
            pltpu.sync_copy(rows_v.at[pl.ds(0, 128)],
                            acc.at[cidx.at[2 * jb]], add=True)
            pltpu.sync_copy(rows_v.at[pl.ds(128, 128)],
                            acc.at[cidx.at[2 * jb + 1]], add=True)
            return c2

        return lax.fori_loop(0, GC // 2, body, carry)

    lax.fori_loop(0, KB // GC, super_block, 0)
    plsc.subcore_barrier()
    pltpu.sync_copy(acc.at[pl.ds(s * RPT, RPT)],
                    out2.at[c, pl.ds(s * RPT, RPT)])


def _conv_agg(rowf, col3, ew3, y, zrows):
    return pl.kernel(
        _conv_body,
        out_type=jax.ShapeDtypeStruct((NC, NP, D), F32),
        mesh=_mesh(),
        compiler_params=_SC_PARAMS,
        scratch_types=[
            pltpu.VMEM((EW_,), jnp.int32),
            pltpu.VMEM((GC, 128), jnp.int32),
            pltpu.VMEM((GC, 128), F32),
            pltpu.VMEM((BS, D), F32),
            pltpu.VMEM_SHARED((NP, D), F32),
            pltpu.SemaphoreType.DMA,
        ],
    )(rowf, col3, ew3, y, zrows)


def _mm_body(h_ref, w1c, b1c, wg1, huv_ref, xw_ref):
    hb = h_ref[...]
    huv_ref[...] = jnp.dot(hb, w1c[...], preferred_element_type=F32) + b1c[...]
    xw_ref[...] = jnp.dot(hb, wg1[...], preferred_element_type=F32)


def _node_mm(h, w1c, b1c, wg1):
    rb = 1024
    grid = NP // rb
    full = lambda shp: pl.BlockSpec(shp, lambda i: (0, 0))
    return pl.pallas_call(
        _mm_body,
        grid=grid,
        in_specs=[
            pl.BlockSpec((rb, D), lambda i: (i, 0)),
            full((D, D)), full((1, D)), full((D, D)),
        ],
        out_specs=[
            pl.BlockSpec((rb, D), lambda i: (i, 0)),
            pl.BlockSpec((rb, D), lambda i: (i, 0)),
        ],
        out_shape=[
            jax.ShapeDtypeStruct((NP, D), F32),
            jax.ShapeDtypeStruct((NP, D), F32),
        ],
    )(h, w1c, b1c, wg1)


def _ew_body(q_ref, w2, b2r, ew_ref):
    i = pl.program_id(0)
    z = jnp.maximum(q_ref[...], 0.0)
    t = jnp.dot(z, w2[...], preferred_element_type=F32) + b2r[...]
    eidx = i * 4096 + lax.broadcasted_iota(jnp.int32, (4096, 1), 0)
    ew_ref[...] = jnp.where(eidx < E, jax.nn.sigmoid(t), 0.0)


def _edge_w(q2, w2, b2r):
    grid = EP // 4096
    return pl.pallas_call(
        _ew_body,
        grid=grid,
        in_specs=[
            pl.BlockSpec((4096, 64), lambda i: (i, 0)),
            pl.BlockSpec((64, 1), lambda i: (0, 0)),
            pl.BlockSpec((1, 1), lambda i: (0, 0)),
        ],
        out_specs=pl.BlockSpec((4096, 1), lambda i: (i, 0)),
        out_shape=jax.ShapeDtypeStruct((EP, 1), F32),
    )(q2, w2, b2r)


def _dinv_body(degp_ref, xw_ref, dinv_ref, y1_ref):
    i = pl.program_id(0)
    deg = jnp.sum(degp_ref[...], axis=0)[:, None] + 1.0
    nidx = i * 1024 + lax.broadcasted_iota(jnp.int32, (1024, 1), 0)
    dinv = jnp.where(nidx < N, lax.rsqrt(deg), 0.0)
    dinv_ref[...] = dinv
    y1_ref[...] = dinv * xw_ref[...]


def _dinv_y1(degp, xw1):
    grid = NP // 1024
    return pl.pallas_call(
        _dinv_body,
        grid=grid,
        in_specs=[
            pl.BlockSpec((NW, 1024), lambda i: (0, i)),
            pl.BlockSpec((1024, D), lambda i: (i, 0)),
        ],
        out_specs=[
            pl.BlockSpec((1024, 1), lambda i: (i, 0)),
            pl.BlockSpec((1024, D), lambda i: (i, 0)),
        ],
        out_shape=[
            jax.ShapeDtypeStruct((NP, 1), F32),
            jax.ShapeDtypeStruct((NP, D), F32),
        ],
    )(degp, xw1)


def _post_body(p_ref, dinv_ref, xw_ref, bg_ref, wg2_ref, xw2_ref, y2_ref):
    dinv = dinv_ref[...]
    acc = p_ref[0] + p_ref[1]
    h1 = jnp.maximum(dinv * acc + dinv * dinv * xw_ref[...] + bg_ref[...], 0.0)
    xw2 = jnp.dot(h1, wg2_ref[...], preferred_element_type=F32)
    xw2_ref[...] = xw2
    y2_ref[...] = dinv * xw2


def _post_conv1(p, dinv, xw1, bgr, wg2):
    grid = NP // 1024
    return pl.pallas_call(
        _post_body,
        grid=grid,
        in_specs=[
            pl.BlockSpec((NC, 1024, D), lambda i: (0, i, 0)),
            pl.BlockSpec((1024, 1), lambda i: (i, 0)),
            pl.BlockSpec((1024, D), lambda i: (i, 0)),
            pl.BlockSpec((1, D), lambda i: (0, 0)),
            pl.BlockSpec((D, D), lambda i: (0, 0)),
        ],
        out_specs=[
            pl.BlockSpec((1024, D), lambda i: (i, 0)),
            pl.BlockSpec((1024, D), lambda i: (i, 0)),
        ],
        out_shape=[
            jax.ShapeDtypeStruct((NP, D), F32),
            jax.ShapeDtypeStruct((NP, D), F32),
        ],
    )(p, dinv, xw1, bgr, wg2)


def _final_body(p_ref, dinv_ref, xw2_ref, bg2_ref, wfc_ref, bfc_ref, out_ref):
    dinv = dinv_ref[...]
    acc = p_ref[0] + p_ref[1]
    h2 = jnp.maximum(dinv * acc + dinv * dinv * xw2_ref[...] + bg2_ref[...],
                     0.0)
    nidx = lax.broadcasted_iota(jnp.int32, (NP, 1), 0)
    pooled = jnp.sum(jnp.where(nidx < N, h2, 0.0), axis=0, keepdims=True) / N
    out_ref[...] = jax.nn.sigmoid(
        jnp.dot(pooled, wfc_ref[...], preferred_element_type=F32)
        + bfc_ref[...])


def _final(p2, dinv, xw2, bg2r, wfc, bfcr):
    return pl.pallas_call(
        _final_body,
        out_shape=jax.ShapeDtypeStruct((1, 1), F32),
    )(p2, dinv, xw2, bg2r, wfc, bfcr)


def kernel(x, edge_index, emb_table, W1, b1, W2, b2, Wg1, bg1, Wg2, bg2,
           Wfc, bfc):
    x = x.astype(jnp.int32)
    row = edge_index[0].astype(jnp.int32)
    col = edge_index[1].astype(jnp.int32)
    x2 = jnp.concatenate([x, jnp.zeros((NP - N,), jnp.int32)]).reshape(
        NB_N, 128)
    zpad = jnp.zeros((EP - E,), jnp.int32)
    rowf = jnp.concatenate([row, zpad]).reshape(NW, EW_)
    colf = jnp.concatenate([col, zpad]).reshape(NW, EW_)
    col3 = colf.reshape(NW, KB, 128)
    zn = jnp.zeros((NP,), F32)
    zrows = jnp.zeros((NP, D), F32)

    h = _emb_gather(x2, emb_table)
    w1c = jnp.concatenate([W1[:D], W1[D:]], axis=1)
    b1c = jnp.concatenate([jnp.zeros((64,), F32), b1]).reshape(1, D)
    huv, xw1 = _node_mm(h, w1c, b1c, Wg1)
    q4 = _edge_q(rowf, colf, huv)
    ew2 = _edge_w(q4.reshape(EP, 64), W2, b2.reshape(1, 1))
    ew3 = ew2.reshape(NW, KB, 128)
    degp = _deg_partials(col3, ew3, zn)
    dinv, y1 = _dinv_y1(degp, xw1)
    p1 = _conv_agg(rowf, col3, ew3, y1, zrows)
    xw2, y2 = _post_conv1(p1, dinv, xw1, bg1.reshape(1, D), Wg2)
    p2 = _conv_agg(rowf, col3, ew3, y2, zrows)
    out = _final(p2, dinv, xw2, bg2.reshape(1, D), Wfc, bfc.reshape(1, 1))
    return out.reshape(1)


# speedup vs baseline: 1.6210x; 1.3232x over previous
import functools

import jax
import jax.numpy as jnp
from jax import lax
from jax.experimental import pallas as pl
from jax.experimental.pallas import tpu as pltpu
from jax.experimental.pallas import tpu_sc as plsc

N, E, V, D = 10000, 320000, 100000, 128
NC, NS, LN = 2, 16, 16
NW = NC * NS
NP = 10240
NB_N = NP // 128
EP = 323584
KB = EP // (NW * 128)
EW_ = KB * 128
BS = 256
NBL = EW_ // BS
GC = 16
RPT = NP // NS
F32 = jnp.float32

_mesh = functools.partial(
    plsc.VectorSubcoreMesh, core_axis_name="c", subcore_axis_name="s")
_SC_PARAMS = pltpu.CompilerParams(needs_layout_passes=False)


def _wid():
    return lax.axis_index("c") * NS + lax.axis_index("s")


def _emb_body(x2, emb, h_out, idx_v, rows_v, sem):
    w = _wid()
    for t in range(3):
        b = w + t * NW

        @pl.when(b < NB_N)
        def _():
            pltpu.sync_copy(x2.at[b], idx_v)
            pltpu.async_copy(emb.at[idx_v], rows_v, sem).wait()
            pltpu.sync_copy(rows_v, h_out.at[pl.ds(b * 128, 128)])


def _emb_gather(x2, emb):
    return pl.kernel(
        _emb_body,
        out_type=jax.ShapeDtypeStruct((NP, D), F32),
        mesh=_mesh(),
        compiler_params=_SC_PARAMS,
        scratch_types=[
            pltpu.VMEM((128,), jnp.int32),
            pltpu.VMEM((128, D), F32),
            pltpu.SemaphoreType.DMA,
        ],
    )(x2, emb)


def _edgeq_body(row3, col3, huv, q4, ridx, cidx, u_v, v_v, q_v, sem):
    w = _wid()
    pltpu.sync_copy(row3.at[w], ridx)
    pltpu.sync_copy(col3.at[w], cidx)

    def body(j, carry):
        cp1 = pltpu.async_copy(huv.at[ridx.at[j]], u_v, sem)
        cp2 = pltpu.async_copy(huv.at[cidx.at[j]], v_v, sem)
        cp1.wait()
        cp2.wait()

        def addhalf(e, c2):
            for k in range(64 // LN):
                q_v[e, pl.ds(k * LN, LN)] = (
                    u_v[e, pl.ds(k * LN, LN)]
                    + v_v[e, pl.ds(64 + k * LN, LN)])
            return c2

        lax.fori_loop(0, 128, addhalf, 0)
        pltpu.sync_copy(q_v, q4.at[w, j])
        return carry

    lax.fori_loop(0, KB, body, 0)


def _edge_q(row3, col3, huv):
    return pl.kernel(
        _edgeq_body,
        out_type=jax.ShapeDtypeStruct((NW, KB, 128, 64), F32),
        mesh=_mesh(),
        compiler_params=_SC_PARAMS,
        scratch_types=[
            pltpu.VMEM((KB, 128), jnp.int32),
            pltpu.VMEM((KB, 128), jnp.int32),
            pltpu.VMEM((128, D), F32),
            pltpu.VMEM((128, D), F32),
            pltpu.VMEM((128, 64), F32),
            pltpu.SemaphoreType.DMA,
        ],
    )(row3, col3, huv)


def _deg_body(col3, ew3, zn, degp, cidx, ew_v, deg_v):
    w = _wid()
    pltpu.sync_copy(zn, deg_v)
    pltpu.sync_copy(col3.at[w], cidx)
    pltpu.sync_copy(ew3.at[w], ew_v)

    def body(j, carry):
        def inner(g, c2):
            idx16 = cidx[j, pl.ds(g * LN, LN)]
            w16 = ew_v[j, pl.ds(g * LN, LN)]
            plsc.addupdate_scatter(deg_v, [idx16], w16)
            return c2
        return lax.fori_loop(0, 128 // LN, inner, carry)

    lax.fori_loop(0, KB, body, 0)
    pltpu.sync_copy(deg_v, degp.at[w])


def _deg_partials(col3, ew3, zn):
    return pl.kernel(
        _deg_body,
        out_type=jax.ShapeDtypeStruct((NW, NP), F32),
        mesh=_mesh(),
        compiler_params=_SC_PARAMS,
        scratch_types=[
            pltpu.VMEM((KB, 128), jnp.int32),
            pltpu.VMEM((KB, 128), F32),
            pltpu.VMEM((NP,), F32),
        ],
    )(col3, ew3, zn)


def _conv_body(row3, col3, ew3, y, zrows, out2, ridx, cidx, ew_v, rows_v,
               acc, sem):
    c = lax.axis_index("c")
    s = lax.axis_index("s")
    w = c * NS + s
    pltpu.sync_copy(zrows.at[pl.ds(s * RPT, RPT)], acc.at[pl.ds(s * RPT, RPT)])
    pltpu.sync_copy(row3.at[w], ridx)
    pltpu.sync_copy(col3.at[w], cidx)
    pltpu.sync_copy(ew3.at[w], ew_v)
    plsc.subcore_barrier()

    def body(j, carry):
        pltpu.async_copy(y.at[ridx.at[j]], rows_v, sem).wait()
        jsp = jnp.full((LN,), j, jnp.int32)

        def scale(e, c2):
            esp = jnp.full((LN,), e, jnp.int32)
            wsp = plsc.load_gather(ew_v, [jsp, esp])
            for k in range(D // LN):
                rows_v[e, pl.ds(k * LN, LN)] = (
                    rows_v[e, pl.ds(k * LN, LN)] * wsp)
            return c2

        lax.fori_loop(0, 128, scale, 0)
        pltpu.sync_copy(rows_v, acc.at[cidx.at[j]], add=True)
        return carry

    lax.fori_loop(0, KB, body, 0)
    plsc.subcore_barrier()
    pltpu.sync_copy(acc.at[pl.ds(s * RPT, RPT)],
                    out2.at[c, pl.ds(s * RPT, RPT)])


def _conv_agg(row3, col3, ew3, y, zrows):
    return pl.kernel(
        _conv_body,
        out_type=jax.ShapeDtypeStruct((NC, NP, D), F32),
        mesh=_mesh(),
        compiler_params=_SC_PARAMS,
        scratch_types=[
            pltpu.VMEM((KB, 128), jnp.int32),
            pltpu.VMEM((KB, 128), jnp.int32),
            pltpu.VMEM((KB, 128), F32),
            pltpu.VMEM((128, D), F32),
            pltpu.VMEM_SHARED((NP, D), F32),
            pltpu.SemaphoreType.DMA,
        ],
    )(row3, col3, ew3, y, zrows)


def _mm_body(h_ref, w1c, b1c, wg1, huv_ref, xw_ref):
    hb = h_ref[...]
    huv_ref[...] = jnp.dot(hb, w1c[...], preferred_element_type=F32) + b1c[...]
    xw_ref[...] = jnp.dot(hb, wg1[...], preferred_element_type=F32)


def _node_mm(h, w1c, b1c, wg1):
    rb = 1024
    grid = NP // rb
    full = lambda shp: pl.BlockSpec(shp, lambda i: (0, 0))
    return pl.pallas_call(
        _mm_body,
        grid=grid,
        in_specs=[
            pl.BlockSpec((rb, D), lambda i: (i, 0)),
            full((D, D)), full((1, D)), full((D, D)),
        ],
        out_specs=[
            pl.BlockSpec((rb, D), lambda i: (i, 0)),
            pl.BlockSpec((rb, D), lambda i: (i, 0)),
        ],
        out_shape=[
            jax.ShapeDtypeStruct((NP, D), F32),
            jax.ShapeDtypeStruct((NP, D), F32),
        ],
    )(h, w1c, b1c, wg1)


def _ew_body(q_ref, w2, b2r, ew_ref):
    i = pl.program_id(0)
    z = jnp.maximum(q_ref[...], 0.0)
    t = jnp.dot(z, w2[...], preferred_element_type=F32) + b2r[...]
    eidx = i * 4096 + lax.broadcasted_iota(jnp.int32, (4096, 1), 0)
    ew_ref[...] = jnp.where(eidx < E, jax.nn.sigmoid(t), 0.0)


def _edge_w(q2, w2, b2r):
    grid = EP // 4096
    return pl.pallas_call(
        _ew_body,
        grid=grid,
        in_specs=[
            pl.BlockSpec((4096, 64), lambda i: (i, 0)),
            pl.BlockSpec((64, 1), lambda i: (0, 0)),
            pl.BlockSpec((1, 1), lambda i: (0, 0)),
        ],
        out_specs=pl.BlockSpec((4096, 1), lambda i: (i, 0)),
        out_shape=jax.ShapeDtypeStruct((EP, 1), F32),
    )(q2, w2, b2r)


def _dinv_body(degp_ref, xw_ref, dinv_ref, y1_ref):
    i = pl.program_id(0)
    deg = jnp.sum(degp_ref[...], axis=0)[:, None] + 1.0
    nidx = i * 1024 + lax.broadcasted_iota(jnp.int32, (1024, 1), 0)
    dinv = jnp.where(nidx < N, lax.rsqrt(deg), 0.0)
    dinv_ref[...] = dinv
    y1_ref[...] = dinv * xw_ref[...]


def _dinv_y1(degp, xw1):
    grid = NP // 1024
    return pl.pallas_call(
        _dinv_body,
        grid=grid,
        in_specs=[
            pl.BlockSpec((NW, 1024), lambda i: (0, i)),
            pl.BlockSpec((1024, D), lambda i: (i, 0)),
        ],
        out_specs=[
            pl.BlockSpec((1024, 1), lambda i: (i, 0)),
            pl.BlockSpec((1024, D), lambda i: (i, 0)),
        ],
        out_shape=[
            jax.ShapeDtypeStruct((NP, 1), F32),
            jax.ShapeDtypeStruct((NP, D), F32),
        ],
    )(degp, xw1)


def _post_body(p_ref, dinv_ref, xw_ref, bg_ref, wg2_ref, xw2_ref, y2_ref):
    dinv = dinv_ref[...]
    acc = p_ref[0] + p_ref[1]
    h1 = jnp.maximum(dinv * acc + dinv * dinv * xw_ref[...] + bg_ref[...], 0.0)
    xw2 = jnp.dot(h1, wg2_ref[...], preferred_element_type=F32)
    xw2_ref[...] = xw2
    y2_ref[...] = dinv * xw2


def _post_conv1(p, dinv, xw1, bgr, wg2):
    grid = NP // 1024
    return pl.pallas_call(
        _post_body,
        grid=grid,
        in_specs=[
            pl.BlockSpec((NC, 1024, D), lambda i: (0, i, 0)),
            pl.BlockSpec((1024, 1), lambda i: (i, 0)),
            pl.BlockSpec((1024, D), lambda i: (i, 0)),
            pl.BlockSpec((1, D), lambda i: (0, 0)),
            pl.BlockSpec((D, D), lambda i: (0, 0)),
        ],
        out_specs=[
            pl.BlockSpec((1024, D), lambda i: (i, 0)),
            pl.BlockSpec((1024, D), lambda i: (i, 0)),
        ],
        out_shape=[
            jax.ShapeDtypeStruct((NP, D), F32),
            jax.ShapeDtypeStruct((NP, D), F32),
        ],
    )(p, dinv, xw1, bgr, wg2)


def _final_body(p_ref, dinv_ref, xw2_ref, bg2_ref, wfc_ref, bfc_ref, out_ref):
    dinv = dinv_ref[...]
    acc = p_ref[0] + p_ref[1]
    h2 = jnp.maximum(dinv * acc + dinv * dinv * xw2_ref[...] + bg2_ref[...],
                     0.0)
    nidx = lax.broadcasted_iota(jnp.int32, (NP, 1), 0)
    pooled = jnp.sum(jnp.where(nidx < N, h2, 0.0), axis=0, keepdims=True) / N
    out_ref[...] = jax.nn.sigmoid(
        jnp.dot(pooled, wfc_ref[...], preferred_element_type=F32)
        + bfc_ref[...])


def _final(p2, dinv, xw2, bg2r, wfc, bfcr):
    return pl.pallas_call(
        _final_body,
        out_shape=jax.ShapeDtypeStruct((1, 1), F32),
    )(p2, dinv, xw2, bg2r, wfc, bfcr)


def kernel(x, edge_index, emb_table, W1, b1, W2, b2, Wg1, bg1, Wg2, bg2,
           Wfc, bfc):
    x = x.astype(jnp.int32)
    row = edge_index[0].astype(jnp.int32)
    col = edge_index[1].astype(jnp.int32)
    x2 = jnp.concatenate([x, jnp.zeros((NP - N,), jnp.int32)]).reshape(
        NB_N, 128)
    zpad = jnp.zeros((EP - E,), jnp.int32)
    row3 = jnp.concatenate([row, zpad]).reshape(NW, KB, 128)
    col3 = jnp.concatenate([col, zpad]).reshape(NW, KB, 128)
    zn = jnp.zeros((NP,), F32)
    zrows = jnp.zeros((NP, D), F32)

    h = _emb_gather(x2, emb_table)
    w1c = jnp.concatenate([W1[:D], W1[D:]], axis=1)
    b1c = jnp.concatenate([jnp.zeros((64,), F32), b1]).reshape(1, D)
    huv, xw1 = _node_mm(h, w1c, b1c, Wg1)
    q4 = _edge_q(row3, col3, huv)
    ew2 = _edge_w(q4.reshape(EP, 64), W2, b2.reshape(1, 1))
    ew3 = ew2.reshape(NW, KB, 128)
    degp = _deg_partials(col3, ew3, zn)
    dinv, y1 = _dinv_y1(degp, xw1)
    p1 = _conv_agg(row3, col3, ew3, y1, zrows)
    xw2, y2 = _post_conv1(p1, dinv, xw1, bg1.reshape(1, D), Wg2)
    p2 = _conv_agg(row3, col3, ew3, y2, zrows)
    out = _final(p2, dinv, xw2, bg2.reshape(1, D), Wfc, bfc.reshape(1, 1))
    return out.reshape(1)


# speedup vs baseline: 1.6227x; 1.0010x over previous
import functools

import jax
import jax.numpy as jnp
from jax import lax
from jax.experimental import pallas as pl
from jax.experimental.pallas import tpu as pltpu
from jax.experimental.pallas import tpu_sc as plsc

N, E, V, D = 10000, 320000, 100000, 128
NC, NS, LN = 2, 16, 16
NW = NC * NS
NP = 10240
NB_N = NP // 128
EP = 323584
KB = EP // (NW * 128)
EW_ = KB * 128
BS = 256
NBL = EW_ // BS
GC = 16
RPT = NP // NS
F32 = jnp.float32

_mesh = functools.partial(
    plsc.VectorSubcoreMesh, core_axis_name="c", subcore_axis_name="s")
_SC_PARAMS = pltpu.CompilerParams(needs_layout_passes=False)


def _wid():
    return lax.axis_index("c") * NS + lax.axis_index("s")


def _emb_body(x2, emb, h_out, idx_v, rows_v, sem):
    w = _wid()
    for t in range(3):
        b = w + t * NW

        @pl.when(b < NB_N)
        def _():
            pltpu.sync_copy(x2.at[b], idx_v)
            pltpu.async_copy(emb.at[idx_v], rows_v, sem).wait()
            pltpu.sync_copy(rows_v, h_out.at[pl.ds(b * 128, 128)])


def _emb_gather(x2, emb):
    return pl.kernel(
        _emb_body,
        out_type=jax.ShapeDtypeStruct((NP, D), F32),
        mesh=_mesh(),
        compiler_params=_SC_PARAMS,
        scratch_types=[
            pltpu.VMEM((128,), jnp.int32),
            pltpu.VMEM((128, D), F32),
            pltpu.SemaphoreType.DMA,
        ],
    )(x2, emb)


def _edgeq_body(row3, col3, huv, q4, ridx, cidx, u_v, v_v, q_v, sem):
    w = _wid()
    pltpu.sync_copy(row3.at[w], ridx)
    pltpu.sync_copy(col3.at[w], cidx)

    def body(j, carry):
        cp1 = pltpu.async_copy(huv.at[ridx.at[j]], u_v, sem)
        cp2 = pltpu.async_copy(huv.at[cidx.at[j]], v_v, sem)
        cp1.wait()
        cp2.wait()

        def addhalf(e, c2):
            for k in range(64 // LN):
                q_v[e, pl.ds(k * LN, LN)] = (
                    u_v[e, pl.ds(k * LN, LN)]
                    + v_v[e, pl.ds(64 + k * LN, LN)])
            return c2

        lax.fori_loop(0, 128, addhalf, 0)
        pltpu.sync_copy(q_v, q4.at[w, j])
        return carry

    lax.fori_loop(0, KB, body, 0)


def _edge_q(row3, col3, huv):
    return pl.kernel(
        _edgeq_body,
        out_type=jax.ShapeDtypeStruct((NW, KB, 128, 64), F32),
        mesh=_mesh(),
        compiler_params=_SC_PARAMS,
        scratch_types=[
            pltpu.VMEM((KB, 128), jnp.int32),
            pltpu.VMEM((KB, 128), jnp.int32),
            pltpu.VMEM((128, D), F32),
            pltpu.VMEM((128, D), F32),
            pltpu.VMEM((128, 64), F32),
            pltpu.SemaphoreType.DMA,
        ],
    )(row3, col3, huv)


def _deg_body(col3, ew3, zn, degp, cidx, ew_v, deg_v):
    w = _wid()
    pltpu.sync_copy(zn, deg_v)
    pltpu.sync_copy(col3.at[w], cidx)
    pltpu.sync_copy(ew3.at[w], ew_v)

    def body(j, carry):
        def inner(g, c2):
            idx16 = cidx[j, pl.ds(g * LN, LN)]
            w16 = ew_v[j, pl.ds(g * LN, LN)]
            plsc.addupdate_scatter(deg_v, [idx16], w16)
            return c2
        return lax.fori_loop(0, 128 // LN, inner, carry)

    lax.fori_loop(0, KB, body, 0)
    pltpu.sync_copy(deg_v, degp.at[w])


def _deg_partials(col3, ew3, zn):
    return pl.kernel(
        _deg_body,
        out_type=jax.ShapeDtypeStruct((NW, NP), F32),
        mesh=_mesh(),
        compiler_params=_SC_PARAMS,
        scratch_types=[
            pltpu.VMEM((KB, 128), jnp.int32),
            pltpu.VMEM((KB, 128), F32),
            pltpu.VMEM((NP,), F32),
        ],
    )(col3, ew3, zn)


def _conv_body(row3, col3, ewf, y, zrows, out2, ridx, cidx, ew_v, rows_v,
               acc, sem):
    c = lax.axis_index("c")
    s = lax.axis_index("s")
    w = c * NS + s
    pltpu.sync_copy(zrows.at[pl.ds(s * RPT, RPT)], acc.at[pl.ds(s * RPT, RPT)])
    pltpu.sync_copy(row3.at[w], ridx)
    pltpu.sync_copy(col3.at[w], cidx)
    pltpu.sync_copy(ewf.at[w], ew_v)
    plsc.subcore_barrier()

    def body(j, carry):
        pltpu.async_copy(y.at[ridx.at[j]], rows_v, sem).wait()
        base = j * 128

        def scale(e, c2):
            wsp = plsc.load_gather(ew_v, [jnp.full((LN,), base + e,
                                                   jnp.int32)])
            for k in range(D // LN):
                rows_v[e, pl.ds(k * LN, LN)] = (
                    rows_v[e, pl.ds(k * LN, LN)] * wsp)
            return c2

        lax.fori_loop(0, 128, scale, 0)
        pltpu.sync_copy(rows_v, acc.at[cidx.at[j]], add=True)
        return carry

    lax.fori_loop(0, KB, body, 0)
    plsc.subcore_barrier()
    pltpu.sync_copy(acc.at[pl.ds(s * RPT, RPT)],
                    out2.at[c, pl.ds(s * RPT, RPT)])


def _conv_agg(row3, col3, ewf, y, zrows):
    return pl.kernel(
        _conv_body,
        out_type=jax.ShapeDtypeStruct((NC, NP, D), F32),
        mesh=_mesh(),
        compiler_params=_SC_PARAMS,
        scratch_types=[
            pltpu.VMEM((KB, 128), jnp.int32),
            pltpu.VMEM((KB, 128), jnp.int32),
            pltpu.VMEM((EW_,), F32),
            pltpu.VMEM((128, D), F32),
            pltpu.VMEM_SHARED((NP, D), F32),
            pltpu.SemaphoreType.DMA,
        ],
    )(row3, col3, ewf, y, zrows)


def _mm_body(h_ref, w1c, b1c, wg1, huv_ref, xw_ref):
    hb = h_ref[...]
    huv_ref[...] = jnp.dot(hb, w1c[...], preferred_element_type=F32) + b1c[...]
    xw_ref[...] = jnp.dot(hb, wg1[...], preferred_element_type=F32)


def _node_mm(h, w1c, b1c, wg1):
    rb = 1024
    grid = NP // rb
    full = lambda shp: pl.BlockSpec(shp, lambda i: (0, 0))
    return pl.pallas_call(
        _mm_body,
        grid=grid,
        in_specs=[
            pl.BlockSpec((rb, D), lambda i: (i, 0)),
            full((D, D)), full((1, D)), full((D, D)),
        ],
        out_specs=[
            pl.BlockSpec((rb, D), lambda i: (i, 0)),
            pl.BlockSpec((rb, D), lambda i: (i, 0)),
        ],
        out_shape=[
            jax.ShapeDtypeStruct((NP, D), F32),
            jax.ShapeDtypeStruct((NP, D), F32),
        ],
    )(h, w1c, b1c, wg1)


def _ew_body(q_ref, w2, b2r, ew_ref):
    i = pl.program_id(0)
    z = jnp.maximum(q_ref[...], 0.0)
    t = jnp.dot(z, w2[...], preferred_element_type=F32) + b2r[...]
    eidx = i * 4096 + lax.broadcasted_iota(jnp.int32, (4096, 1), 0)
    ew_ref[...] = jnp.where(eidx < E, jax.nn.sigmoid(t), 0.0)


def _edge_w(q2, w2, b2r):
    grid = EP // 4096
    return pl.pallas_call(
        _ew_body,
        grid=grid,
        in_specs=[
            pl.BlockSpec((4096, 64), lambda i: (i, 0)),
            pl.BlockSpec((64, 1), lambda i: (0, 0)),
            pl.BlockSpec((1, 1), lambda i: (0, 0)),
        ],
        out_specs=pl.BlockSpec((4096, 1), lambda i: (i, 0)),
        out_shape=jax.ShapeDtypeStruct((EP, 1), F32),
    )(q2, w2, b2r)


def _dinv_body(degp_ref, xw_ref, dinv_ref, y1_ref):
    i = pl.program_id(0)
    deg = jnp.sum(degp_ref[...], axis=0)[:, None] + 1.0
    nidx = i * 1024 + lax.broadcasted_iota(jnp.int32, (1024, 1), 0)
    dinv = jnp.where(nidx < N, lax.rsqrt(deg), 0.0)
    dinv_ref[...] = dinv
    y1_ref[...] = dinv * xw_ref[...]


def _dinv_y1(degp, xw1):
    grid = NP // 1024
    return pl.pallas_call(
        _dinv_body,
        grid=grid,
        in_specs=[
            pl.BlockSpec((NW, 1024), lambda i: (0, i)),
            pl.BlockSpec((1024, D), lambda i: (i, 0)),
        ],
        out_specs=[
            pl.BlockSpec((1024, 1), lambda i: (i, 0)),
            pl.BlockSpec((1024, D), lambda i: (i, 0)),
        ],
        out_shape=[
            jax.ShapeDtypeStruct((NP, 1), F32),
            jax.ShapeDtypeStruct((NP, D), F32),
        ],
    )(degp, xw1)


def _post_body(p_ref, dinv_ref, xw_ref, bg_ref, wg2_ref, xw2_ref, y2_ref):
    dinv = dinv_ref[...]
    acc = p_ref[0] + p_ref[1]
    h1 = jnp.maximum(dinv * acc + dinv * dinv * xw_ref[...] + bg_ref[...], 0.0)
    xw2 = jnp.dot(h1, wg2_ref[...], preferred_element_type=F32)
    xw2_ref[...] = xw2
    y2_ref[...] = dinv * xw2


def _post_conv1(p, dinv, xw1, bgr, wg2):
    grid = NP // 1024
    return pl.pallas_call(
        _post_body,
        grid=grid,
        in_specs=[
            pl.BlockSpec((NC, 1024, D), lambda i: (0, i, 0)),
            pl.BlockSpec((1024, 1), lambda i: (i, 0)),
            pl.BlockSpec((1024, D), lambda i: (i, 0)),
            pl.BlockSpec((1, D), lambda i: (0, 0)),
            pl.BlockSpec((D, D), lambda i: (0, 0)),
        ],
        out_specs=[
            pl.BlockSpec((1024, D), lambda i: (i, 0)),
            pl.BlockSpec((1024, D), lambda i: (i, 0)),
        ],
        out_shape=[
            jax.ShapeDtypeStruct((NP, D), F32),
            jax.ShapeDtypeStruct((NP, D), F32),
        ],
    )(p, dinv, xw1, bgr, wg2)


def _final_body(p_ref, dinv_ref, xw2_ref, bg2_ref, wfc_ref, bfc_ref, out_ref):
    dinv = dinv_ref[...]
    acc = p_ref[0] + p_ref[1]
    h2 = jnp.maximum(dinv * acc + dinv * dinv * xw2_ref[...] + bg2_ref[...],
                     0.0)
    nidx = lax.broadcasted_iota(jnp.int32, (NP, 1), 0)
    pooled = jnp.sum(jnp.where(nidx < N, h2, 0.0), axis=0, keepdims=True) / N
    out_ref[...] = jax.nn.sigmoid(
        jnp.dot(pooled, wfc_ref[...], preferred_element_type=F32)
        + bfc_ref[...])


def _final(p2, dinv, xw2, bg2r, wfc, bfcr):
    return pl.pallas_call(
        _final_body,
        out_shape=jax.ShapeDtypeStruct((1, 1), F32),
    )(p2, dinv, xw2, bg2r, wfc, bfcr)


def kernel(x, edge_index, emb_table, W1, b1, W2, b2, Wg1, bg1, Wg2, bg2,
           Wfc, bfc):
    x = x.astype(jnp.int32)
    row = edge_index[0].astype(jnp.int32)
    col = edge_index[1].astype(jnp.int32)
    x2 = jnp.concatenate([x, jnp.zeros((NP - N,), jnp.int32)]).reshape(
        NB_N, 128)
    zpad = jnp.zeros((EP - E,), jnp.int32)
    row3 = jnp.concatenate([row, zpad]).reshape(NW, KB, 128)
    col3 = jnp.concatenate([col, zpad]).reshape(NW, KB, 128)
    zn = jnp.zeros((NP,), F32)
    zrows = jnp.zeros((NP, D), F32)

    h = _emb_gather(x2, emb_table)
    w1c = jnp.concatenate([W1[:D], W1[D:]], axis=1)
    b1c = jnp.concatenate([jnp.zeros((64,), F32), b1]).reshape(1, D)
    huv, xw1 = _node_mm(h, w1c, b1c, Wg1)
    q4 = _edge_q(row3, col3, huv)
    ew2 = _edge_w(q4.reshape(EP, 64), W2, b2.reshape(1, 1))
    ew3 = ew2.reshape(NW, KB, 128)
    degp = _deg_partials(col3, ew3, zn)
    dinv, y1 = _dinv_y1(degp, xw1)
    ewfr = ew2.reshape(NW, EW_)
    p1 = _conv_agg(row3, col3, ewfr, y1, zrows)
    xw2, y2 = _post_conv1(p1, dinv, xw1, bg1.reshape(1, D), Wg2)
    p2 = _conv_agg(row3, col3, ewfr, y2, zrows)
    out = _final(p2, dinv, xw2, bg2.reshape(1, D), Wfc, bfc.reshape(1, 1))
    return out.reshape(1)


# speedup vs baseline: 1.7155x; 1.0572x over previous
import functools

import jax
import jax.numpy as jnp
from jax import lax
from jax.experimental import pallas as pl
from jax.experimental.pallas import tpu as pltpu
from jax.experimental.pallas import tpu_sc as plsc

N, E, V, D = 10000, 320000, 100000, 128
NC, NS, LN = 2, 16, 16
NW = NC * NS
NP = 10240
NB_N = NP // 128
EP = 323584
KB = EP // (NW * 128)
EW_ = KB * 128
BS = 256
NBL = EW_ // BS
GC = 16
RPT = NP // NS
F32 = jnp.float32

_mesh = functools.partial(
    plsc.VectorSubcoreMesh, core_axis_name="c", subcore_axis_name="s")
_SC_PARAMS = pltpu.CompilerParams(needs_layout_passes=False)


def _wid():
    return lax.axis_index("c") * NS + lax.axis_index("s")


def _emb_body(x2, emb, h_out, idx_v, rows_v, sem):
    w = _wid()
    for t in range(3):
        b = w + t * NW

        @pl.when(b < NB_N)
        def _():
            pltpu.sync_copy(x2.at[b], idx_v)
            pltpu.async_copy(emb.at[idx_v], rows_v, sem).wait()
            pltpu.sync_copy(rows_v, h_out.at[pl.ds(b * 128, 128)])


def _emb_gather(x2, emb):
    return pl.kernel(
        _emb_body,
        out_type=jax.ShapeDtypeStruct((NP, D), F32),
        mesh=_mesh(),
        compiler_params=_SC_PARAMS,
        scratch_types=[
            pltpu.VMEM((128,), jnp.int32),
            pltpu.VMEM((128, D), F32),
            pltpu.SemaphoreType.DMA,
        ],
    )(x2, emb)


def _edgeq_body(row3, col3, huv, q4, ridx, cidx, u_v, v_v, q_v, sem):
    w = _wid()
    pltpu.sync_copy(row3.at[w], ridx)
    pltpu.sync_copy(col3.at[w], cidx)

    def body(j, carry):
        cp1 = pltpu.async_copy(huv.at[ridx.at[j]], u_v, sem)
        cp2 = pltpu.async_copy(huv.at[cidx.at[j]], v_v, sem)
        cp1.wait()
        cp2.wait()

        @plsc.parallel_loop(0, 128)
        def _(e):
            for k in range(64 // LN):
                q_v[e, pl.ds(k * LN, LN)] = (
                    u_v[e, pl.ds(k * LN, LN)]
                    + v_v[e, pl.ds(64 + k * LN, LN)])
        pltpu.sync_copy(q_v, q4.at[w, j])
        return carry

    lax.fori_loop(0, KB, body, 0)


def _edge_q(row3, col3, huv):
    return pl.kernel(
        _edgeq_body,
        out_type=jax.ShapeDtypeStruct((NW, KB, 128, 64), F32),
        mesh=_mesh(),
        compiler_params=_SC_PARAMS,
        scratch_types=[
            pltpu.VMEM((KB, 128), jnp.int32),
            pltpu.VMEM((KB, 128), jnp.int32),
            pltpu.VMEM((128, D), F32),
            pltpu.VMEM((128, D), F32),
            pltpu.VMEM((128, 64), F32),
            pltpu.SemaphoreType.DMA,
        ],
    )(row3, col3, huv)


def _deg_body(col3, ew3, zn, degp, cidx, ew_v, deg_v):
    w = _wid()
    pltpu.sync_copy(zn, deg_v)
    pltpu.sync_copy(col3.at[w], cidx)
    pltpu.sync_copy(ew3.at[w], ew_v)

    def body(j, carry):
        def inner(g, c2):
            idx16 = cidx[j, pl.ds(g * LN, LN)]
            w16 = ew_v[j, pl.ds(g * LN, LN)]
            plsc.addupdate_scatter(deg_v, [idx16], w16)
            return c2
        return lax.fori_loop(0, 128 // LN, inner, carry)

    lax.fori_loop(0, KB, body, 0)
    pltpu.sync_copy(deg_v, degp.at[w])


def _deg_partials(col3, ew3, zn):
    return pl.kernel(
        _deg_body,
        out_type=jax.ShapeDtypeStruct((NW, NP), F32),
        mesh=_mesh(),
        compiler_params=_SC_PARAMS,
        scratch_types=[
            pltpu.VMEM((KB, 128), jnp.int32),
            pltpu.VMEM((KB, 128), F32),
            pltpu.VMEM((NP,), F32),
        ],
    )(col3, ew3, zn)


def _conv_body(row3, col3, ewf, y, zrows, out2, ridx, cidx, ew_v, rows_v,
               acc, sem):
    c = lax.axis_index("c")
    s = lax.axis_index("s")
    w = c * NS + s
    pltpu.sync_copy(zrows.at[pl.ds(s * RPT, RPT)], acc.at[pl.ds(s * RPT, RPT)])
    pltpu.sync_copy(row3.at[w], ridx)
    pltpu.sync_copy(col3.at[w], cidx)
    pltpu.sync_copy(ewf.at[w], ew_v)
    plsc.subcore_barrier()

    def body(j, carry):
        pltpu.async_copy(y.at[ridx.at[j]], rows_v, sem).wait()
        base = j * 128

        @plsc.parallel_loop(0, 128)
        def _(e):
            wsp = plsc.load_gather(ew_v, [jnp.full((LN,), base + e,
                                                   jnp.int32)])
            for k in range(D // LN):
                rows_v[e, pl.ds(k * LN, LN)] = (
                    rows_v[e, pl.ds(k * LN, LN)] * wsp)
        pltpu.sync_copy(rows_v, acc.at[cidx.at[j]], add=True)
        return carry

    lax.fori_loop(0, KB, body, 0)
    plsc.subcore_barrier()
    pltpu.sync_copy(acc.at[pl.ds(s * RPT, RPT)],
                    out2.at[c, pl.ds(s * RPT, RPT)])


def _conv_agg(row3, col3, ewf, y, zrows):
    return pl.kernel(
        _conv_body,
        out_type=jax.ShapeDtypeStruct((NC, NP, D), F32),
        mesh=_mesh(),
        compiler_params=_SC_PARAMS,
        scratch_types=[
            pltpu.VMEM((KB, 128), jnp.int32),
            pltpu.VMEM((KB, 128), jnp.int32),
            pltpu.VMEM((EW_,), F32),
            pltpu.VMEM((128, D), F32),
            pltpu.VMEM_SHARED((NP, D), F32),
            pltpu.SemaphoreType.DMA,
        ],
    )(row3, col3, ewf, y, zrows)


def _mm_body(h_ref, w1c, b1c, wg1, huv_ref, xw_ref):
    hb = h_ref[...]
    huv_ref[...] = jnp.dot(hb, w1c[...], preferred_element_type=F32) + b1c[...]
    xw_ref[...] = jnp.dot(hb, wg1[...], preferred_element_type=F32)


def _node_mm(h, w1c, b1c, wg1):
    rb = 1024
    grid = NP // rb
    full = lambda shp: pl.BlockSpec(shp, lambda i: (0, 0))
    return pl.pallas_call(
        _mm_body,
        grid=grid,
        in_specs=[
            pl.BlockSpec((rb, D), lambda i: (i, 0)),
            full((D, D)), full((1, D)), full((D, D)),
        ],
        out_specs=[
            pl.BlockSpec((rb, D), lambda i: (i, 0)),
            pl.BlockSpec((rb, D), lambda i: (i, 0)),
        ],
        out_shape=[
            jax.ShapeDtypeStruct((NP, D), F32),
            jax.ShapeDtypeStruct((NP, D), F32),
        ],
    )(h, w1c, b1c, wg1)


def _ew_body(q_ref, w2, b2r, ew_ref):
    i = pl.program_id(0)
    z = jnp.maximum(q_ref[...], 0.0)
    t = jnp.dot(z, w2[...], preferred_element_type=F32) + b2r[...]
    eidx = i * 4096 + lax.broadcasted_iota(jnp.int32, (4096, 1), 0)
    ew_ref[...] = jnp.where(eidx < E, jax.nn.sigmoid(t), 0.0)


def _edge_w(q2, w2, b2r):
    grid = EP // 4096
    return pl.pallas_call(
        _ew_body,
        grid=grid,
        in_specs=[
            pl.BlockSpec((4096, 64), lambda i: (i, 0)),
            pl.BlockSpec((64, 1), lambda i: (0, 0)),
            pl.BlockSpec((1, 1), lambda i: (0, 0)),
        ],
        out_specs=pl.BlockSpec((4096, 1), lambda i: (i, 0)),
        out_shape=jax.ShapeDtypeStruct((EP, 1), F32),
    )(q2, w2, b2r)


def _dinv_body(degp_ref, xw_ref, dinv_ref, y1_ref):
    i = pl.program_id(0)
    deg = jnp.sum(degp_ref[...], axis=0)[:, None] + 1.0
    nidx = i * 1024 + lax.broadcasted_iota(jnp.int32, (1024, 1), 0)
    dinv = jnp.where(nidx < N, lax.rsqrt(deg), 0.0)
    dinv_ref[...] = dinv
    y1_ref[...] = dinv * xw_ref[...]


def _dinv_y1(degp, xw1):
    grid = NP // 1024
    return pl.pallas_call(
        _dinv_body,
        grid=grid,
        in_specs=[
            pl.BlockSpec((NW, 1024), lambda i: (0, i)),
            pl.BlockSpec((1024, D), lambda i: (i, 0)),
        ],
        out_specs=[
            pl.BlockSpec((1024, 1), lambda i: (i, 0)),
            pl.BlockSpec((1024, D), lambda i: (i, 0)),
        ],
        out_shape=[
            jax.ShapeDtypeStruct((NP, 1), F32),
            jax.ShapeDtypeStruct((NP, D), F32),
        ],
    )(degp, xw1)


def _post_body(p_ref, dinv_ref, xw_ref, bg_ref, wg2_ref, xw2_ref, y2_ref):
    dinv = dinv_ref[...]
    acc = p_ref[0] + p_ref[1]
    h1 = jnp.maximum(dinv * acc + dinv * dinv * xw_ref[...] + bg_ref[...], 0.0)
    xw2 = jnp.dot(h1, wg2_ref[...], preferred_element_type=F32)
    xw2_ref[...] = xw2
    y2_ref[...] = dinv * xw2


def _post_conv1(p, dinv, xw1, bgr, wg2):
    grid = NP // 1024
    return pl.pallas_call(
        _post_body,
        grid=grid,
        in_specs=[
            pl.BlockSpec((NC, 1024, D), lambda i: (0, i, 0)),
            pl.BlockSpec((1024, 1), lambda i: (i, 0)),
            pl.BlockSpec((1024, D), lambda i: (i, 0)),
            pl.BlockSpec((1, D), lambda i: (0, 0)),
            pl.BlockSpec((D, D), lambda i: (0, 0)),
        ],
        out_specs=[
            pl.BlockSpec((1024, D), lambda i: (i, 0)),
            pl.BlockSpec((1024, D), lambda i: (i, 0)),
        ],
        out_shape=[
            jax.ShapeDtypeStruct((NP, D), F32),
            jax.ShapeDtypeStruct((NP, D), F32),
        ],
    )(p, dinv, xw1, bgr, wg2)


def _final_body(p_ref, dinv_ref, xw2_ref, bg2_ref, wfc_ref, bfc_ref, out_ref):
    dinv = dinv_ref[...]
    acc = p_ref[0] + p_ref[1]
    h2 = jnp.maximum(dinv * acc + dinv * dinv * xw2_ref[...] + bg2_ref[...],
                     0.0)
    nidx = lax.broadcasted_iota(jnp.int32, (NP, 1), 0)
    pooled = jnp.sum(jnp.where(nidx < N, h2, 0.0), axis=0, keepdims=True) / N
    out_ref[...] = jax.nn.sigmoid(
        jnp.dot(pooled, wfc_ref[...], preferred_element_type=F32)
        + bfc_ref[...])


def _final(p2, dinv, xw2, bg2r, wfc, bfcr):
    return pl.pallas_call(
        _final_body,
        out_shape=jax.ShapeDtypeStruct((1, 1), F32),
    )(p2, dinv, xw2, bg2r, wfc, bfcr)


def kernel(x, edge_index, emb_table, W1, b1, W2, b2, Wg1, bg1, Wg2, bg2,
           Wfc, bfc):
    x = x.astype(jnp.int32)
    row = edge_index[0].astype(jnp.int32)
    col = edge_index[1].astype(jnp.int32)
    x2 = jnp.concatenate([x, jnp.zeros((NP - N,), jnp.int32)]).reshape(
        NB_N, 128)
    zpad = jnp.zeros((EP - E,), jnp.int32)
    row3 = jnp.concatenate([row, zpad]).reshape(NW, KB, 128)
    col3 = jnp.concatenate([col, zpad]).reshape(NW, KB, 128)
    zn = jnp.zeros((NP,), F32)
    zrows = jnp.zeros((NP, D), F32)

    h = _emb_gather(x2, emb_table)
    w1c = jnp.concatenate([W1[:D], W1[D:]], axis=1)
    b1c = jnp.concatenate([jnp.zeros((64,), F32), b1]).reshape(1, D)
    huv, xw1 = _node_mm(h, w1c, b1c, Wg1)
    q4 = _edge_q(row3, col3, huv)
    ew2 = _edge_w(q4.reshape(EP, 64), W2, b2.reshape(1, 1))
    ew3 = ew2.reshape(NW, KB, 128)
    degp = _deg_partials(col3, ew3, zn)
    dinv, y1 = _dinv_y1(degp, xw1)
    ewfr = ew2.reshape(NW, EW_)
    p1 = _conv_agg(row3, col3, ewfr, y1, zrows)
    xw2, y2 = _post_conv1(p1, dinv, xw1, bg1.reshape(1, D), Wg2)
    p2 = _conv_agg(row3, col3, ewfr, y2, zrows)
    out = _final(p2, dinv, xw2, bg2.reshape(1, D), Wfc, bfc.reshape(1, 1))
    return out.reshape(1)
